# baseline (speedup 1.0000x reference)
import functools

import jax
import jax.numpy as jnp
from jax import lax
from jax.experimental import pallas as pl
from jax.experimental.pallas import tpu as pltpu
from jax.experimental.pallas import tpu_sc as plsc

D_MODEL = 1024
VOCAB = 60
NW = 32
CHUNK = 32
NCHUNK = 16
ROWS = NW * NCHUNK * CHUNK


def _build():
  mesh = plsc.VectorSubcoreMesh(core_axis_name="c", subcore_axis_name="s")

  @functools.partial(
      pl.kernel,
      mesh=mesh,
      out_type=jax.ShapeDtypeStruct((ROWS, D_MODEL), jnp.float32),
      scratch_types=[
          pltpu.VMEM((NCHUNK, CHUNK), jnp.int32),
          pltpu.VMEM((CHUNK, D_MODEL), jnp.float32),
          pltpu.VMEM((CHUNK, D_MODEL), jnp.float32),
          pltpu.SemaphoreType.DMA,
          pltpu.SemaphoreType.DMA,
      ],
  )
  def emb(table_hbm, idx_hbm, out_hbm, idx_v, buf0, buf1, sem0, sem1):
    wid = lax.axis_index("s") * 2 + lax.axis_index("c")
    pltpu.sync_copy(idx_hbm.at[pl.ds(wid * NCHUNK, NCHUNK)], idx_v)
    row0 = wid * (NCHUNK * CHUNK)
    bufs = (buf0, buf1)
    sems = (sem0, sem1)
    cp0 = pltpu.async_copy(table_hbm.at[idx_v.at[0]], bufs[0], sems[0])
    for c in range(NCHUNK):
      if c == 0:
        cp = cp0
      cp.wait()
      if c + 1 < NCHUNK:
        nxt = (c + 1) % 2
        cp = pltpu.async_copy(
            table_hbm.at[idx_v.at[c + 1]], bufs[nxt], sems[nxt])
      pltpu.sync_copy(bufs[c % 2],
                      out_hbm.at[pl.ds(row0 + c * CHUNK, CHUNK)])

  return emb


_emb = _build()


def kernel(x_mark, W):
  B, L, _ = x_mark.shape
  idx = x_mark[:, :, 1].reshape(NW * NCHUNK, CHUNK).astype(jnp.int32)
  out = _emb(W, idx)
  return out.reshape(B, L, 1, D_MODEL)


# speedup vs baseline: 1.4485x; 1.4485x over previous
import functools

import jax
import jax.numpy as jnp
from jax import lax
from jax.experimental import pallas as pl
from jax.experimental.pallas import tpu as pltpu
from jax.experimental.pallas import tpu_sc as plsc

D_MODEL = 1024
VOCAB = 60
NW = 32
CHUNK = 32
NCHUNK = 16
ROWS = NW * NCHUNK * CHUNK


def _build():
  mesh = plsc.VectorSubcoreMesh(core_axis_name="c", subcore_axis_name="s")

  @functools.partial(
      pl.kernel,
      mesh=mesh,
      out_type=jax.ShapeDtypeStruct((4, ROWS // 4, 1, D_MODEL), jnp.float32),
      scratch_types=[
          pltpu.VMEM((NCHUNK, CHUNK), jnp.int32),
          pltpu.VMEM((CHUNK, D_MODEL), jnp.float32),
          pltpu.VMEM((CHUNK, D_MODEL), jnp.float32),
          pltpu.SemaphoreType.DMA,
          pltpu.SemaphoreType.DMA,
      ],
  )
  def emb(table_hbm, idx_hbm, out_hbm, idx_v, buf0, buf1, sem0, sem1):
    wid = lax.axis_index("s") * 2 + lax.axis_index("c")
    pltpu.sync_copy(idx_hbm.at[pl.ds(wid * NCHUNK, NCHUNK)], idx_v)
    row0 = wid * (NCHUNK * CHUNK)
    bufs = (buf0, buf1)
    sems = (sem0, sem1)
    cp0 = pltpu.async_copy(table_hbm.at[idx_v.at[0]], bufs[0], sems[0])
    for c in range(NCHUNK):
      if c == 0:
        cp = cp0
      cp.wait()
      if c + 1 < NCHUNK:
        nxt = (c + 1) % 2
        cp = pltpu.async_copy(
            table_hbm.at[idx_v.at[c + 1]], bufs[nxt], sems[nxt])
      r = row0 + c * CHUNK
      pltpu.sync_copy(bufs[c % 2],
                      out_hbm.at[r // 4096, pl.ds(r % 4096, CHUNK), 0])

  return emb


_emb = _build()


def kernel(x_mark, W):
  B, L, _ = x_mark.shape
  idx = x_mark[:, :, 1].reshape(NW * NCHUNK, CHUNK).astype(jnp.int32)
  return _emb(W, idx)
